# baseline (speedup 1.0000x reference)
import functools

import jax
import jax.numpy as jnp
from jax import lax
from jax.experimental import pallas as pl
from jax.experimental.pallas import tpu as pltpu
from jax.experimental.pallas import tpu_sc as plsc

M = 5000
NC, NS = 2, 16
NW = NC * NS
CAUG = 160
CW = 144
K = 125

_pcall = pl.pallas_call


def _sc_segsum(table, sidx, didx, zeros, s_out, s_pad):
    C = table.shape[1]
    dt = table.dtype
    ch = sidx.shape[0] // NW
    stripe = s_pad // NS
    out_tiles = NS if s_out % NS == 0 else 8
    ostripe = s_out // out_tiles
    mesh = plsc.VectorSubcoreMesh(core_axis_name="c", subcore_axis_name="s")

    @functools.partial(
        pl.kernel, mesh=mesh,
        compiler_params=pltpu.CompilerParams(use_tc_tiling_on_sc=False),
        out_type=jax.ShapeDtypeStruct((NC * s_out, CW), dt),
        scratch_types=[
            pltpu.VMEM((ch, K), jnp.int32),
            pltpu.VMEM((ch, K), jnp.int32),
            pltpu.VMEM((K, C), dt),
            pltpu.VMEM((K, C), dt),
            pltpu.VMEM_SHARED((s_pad, C), dt),
            pltpu.SemaphoreType.DMA,
            pltpu.SemaphoreType.DMA,
        ])
    def run(table_h, sidx_h, didx_h, zeros_h, out_h,
            sidx_v, didx_v, rows0, rows1, acc, sem0, sem1):
        c = lax.axis_index("c")
        s = lax.axis_index("s")
        wid = c * NS + s
        pltpu.sync_copy(sidx_h.at[pl.ds(wid * ch, ch)], sidx_v)
        pltpu.sync_copy(didx_h.at[pl.ds(wid * ch, ch)], didx_v)
        pltpu.sync_copy(zeros_h.at[pl.ds(0, stripe)],
                        acc.at[pl.ds(s * stripe, stripe)])
        plsc.subcore_barrier()
        pltpu.async_copy(table_h.at[sidx_v.at[0]], rows0, sem0)

        def step(t, carry):
            j = 2 * t
            pltpu.async_copy(table_h.at[sidx_v.at[j + 1]], rows1, sem1)
            pltpu.make_async_copy(zeros_h.at[pl.ds(0, K)], rows0, sem0).wait()
            pltpu.sync_copy(rows0, acc.at[didx_v.at[j]], add=True)

            @pl.when(t + 1 < ch // 2)
            def _():
                pltpu.async_copy(table_h.at[sidx_v.at[j + 2]], rows0, sem0)

            pltpu.make_async_copy(zeros_h.at[pl.ds(0, K)], rows1, sem1).wait()
            pltpu.sync_copy(rows1, acc.at[didx_v.at[j + 1]], add=True)
            return carry

        lax.fori_loop(0, ch // 2, step, 0)
        plsc.subcore_barrier()

        @pl.when(s < out_tiles)
        def _():
            pltpu.sync_copy(acc.at[pl.ds(s * ostripe, ostripe), pl.ds(0, CW)],
                            out_h.at[pl.ds(c * s_out + s * ostripe, ostripe)])

    return run(table, sidx, didx, zeros)


def _hmask(rows, cols, blk):
    r = lax.broadcasted_iota(jnp.int32, (rows, cols), 0)
    c = lax.broadcasted_iota(jnp.int32, (rows, cols), 1)
    return (r == c // blk).astype(jnp.float32)


def _proj_body(x_ref, w_ref, b_ref, aw_ref, out_ref):
    x = x_ref[...]
    bn = x.shape[0]
    feats, auxs = [], []
    for h in range(4):
        xw = jnp.dot(x, w_ref[h], preferred_element_type=jnp.float32)
        xw = xw + b_ref[h][None, :]
        feats.append(xw)
        auxs.append(jnp.sum(xw * aw_ref[h][None, :], axis=1, keepdims=True))
    ones = jnp.ones((bn, 1), jnp.float32)
    zpad = jnp.zeros((bn, CAUG - 133), jnp.float32)
    out_ref[...] = jnp.concatenate(
        feats + auxs + [ones, zpad], axis=1).astype(jnp.bfloat16)


def _edge_body(p0_ref, p1_ref, out_ref):
    p0 = p0_ref[0].astype(jnp.float32)
    p1 = p1_ref[0].astype(jnp.float32)
    esum = p0[:, :128] + p1[:, :128]
    aux = p0[:, 128:144] + p1[:, 128:144]
    cnt = aux[:, 4:5]
    inv = 1.0 / jnp.maximum(cnt, 1.0)
    alpha128 = jnp.dot(aux, _hmask(16, 128, 32),
                       preferred_element_type=jnp.float32) * inv
    g128 = jnp.exp(jnp.where(alpha128 >= 0.0, alpha128, 0.2 * alpha128))
    yg = g128 * (esum * inv)
    alpha16 = aux * inv
    m16 = (lax.broadcasted_iota(jnp.int32, (1, 16), 1) < 4).astype(jnp.float32)
    g16 = jnp.exp(jnp.where(alpha16 >= 0.0, alpha16, 0.2 * alpha16)) * m16
    out_ref[...] = jnp.concatenate(
        [yg, g16, jnp.zeros((g16.shape[0], CAUG - CW), jnp.float32)],
        axis=1).astype(jnp.bfloat16)


def _final_body(p0_ref, p1_ref, x_ref, convw_ref,
                convb_ref, lng_ref, lnb_ref, gam_ref, out_ref):
    p0 = p0_ref[0].astype(jnp.float32)
    p1 = p1_ref[0].astype(jnp.float32)
    numer = p0[:, :128] + p1[:, :128]
    aux = p0[:, 128:144] + p1[:, 128:144]
    den128 = jnp.dot(aux, _hmask(16, 128, 32),
                     preferred_element_type=jnp.float32) + 1e-12
    o = numer / den128
    o = jnp.where(o > 0.0, o, jnp.exp(jnp.minimum(o, 0.0)) - 1.0)
    mu = jnp.mean(o, axis=1, keepdims=True)
    xc = o - mu
    var = jnp.mean(xc * xc, axis=1, keepdims=True)
    xn = xc * lax.rsqrt(var + 1e-6) * lng_ref[...] + lnb_ref[...]
    xg = 0.5 * xn * (1.0 + lax.erf(xn * 0.7071067811865476))
    xo = jnp.dot(xg, convw_ref[...], preferred_element_type=jnp.float32)
    xo = xo + convb_ref[...]
    out_ref[...] = x_ref[...] + gam_ref[...] * xo


def kernel(X, theta_W, theta_b, atten_e_W, ln_g, ln_b, conv_W, conv_b, gamma,
           v_idx, e_idx):
    N, D = X.shape
    H, _, DH = theta_W.shape
    P = v_idx.shape[0]
    m_pad = ((M + NS * 8 - 1) // (NS * 8)) * (NS * 8)

    f32 = jnp.float32
    bf16 = jnp.bfloat16
    zeros = jnp.zeros((N // NS + 15, CAUG), bf16)
    v2d = v_idx.reshape(P // K, K)
    e2d = e_idx.reshape(P // K, K)

    bn = 1000
    xwaug = _pcall(
        _proj_body,
        grid=(N // bn,),
        in_specs=[
            pl.BlockSpec((bn, D), lambda i: (i, 0)),
            pl.BlockSpec((H, D, DH), lambda i: (0, 0, 0)),
            pl.BlockSpec((H, DH), lambda i: (0, 0)),
            pl.BlockSpec((H, DH), lambda i: (0, 0)),
        ],
        out_specs=pl.BlockSpec((bn, CAUG), lambda i: (i, 0)),
        out_shape=jax.ShapeDtypeStruct((N, CAUG), bf16),
    )(X, theta_W, theta_b, atten_e_W)

    eb = _sc_segsum(xwaug, v2d, e2d, zeros, M, m_pad).reshape(NC, M, CW)

    bm = 1000
    yaug = _pcall(
        _edge_body,
        grid=(M // bm,),
        in_specs=[
            pl.BlockSpec((1, bm, CW), lambda i: (0, i, 0)),
            pl.BlockSpec((1, bm, CW), lambda i: (1, i, 0)),
        ],
        out_specs=pl.BlockSpec((bm, CAUG), lambda i: (i, 0)),
        out_shape=jax.ShapeDtypeStruct((M, CAUG), bf16),
    )(eb, eb)

    vb = _sc_segsum(yaug, e2d, v2d, zeros, N, N).reshape(NC, N, CW)

    out = _pcall(
        _final_body,
        grid=(N // bn,),
        in_specs=[
            pl.BlockSpec((1, bn, CW), lambda i: (0, i, 0)),
            pl.BlockSpec((1, bn, CW), lambda i: (1, i, 0)),
            pl.BlockSpec((bn, D), lambda i: (i, 0)),
            pl.BlockSpec((D, D), lambda i: (0, 0)),
            pl.BlockSpec((1, D), lambda i: (0, 0)),
            pl.BlockSpec((1, D), lambda i: (0, 0)),
            pl.BlockSpec((1, D), lambda i: (0, 0)),
            pl.BlockSpec((1, D), lambda i: (0, 0)),
        ],
        out_specs=pl.BlockSpec((bn, D), lambda i: (i, 0)),
        out_shape=jax.ShapeDtypeStruct((N, D), f32),
    )(vb, vb, X, conv_W, conv_b.reshape(1, D),
      ln_g.reshape(1, D), ln_b.reshape(1, D), gamma.reshape(1, D))
    return out


# speedup vs baseline: 1.0005x; 1.0005x over previous
import functools

import jax
import jax.numpy as jnp
from jax import lax
from jax.experimental import pallas as pl
from jax.experimental.pallas import tpu as pltpu
from jax.experimental.pallas import tpu_sc as plsc

M = 5000
NC, NS = 2, 16
NW = NC * NS
CAUG = 160
CW = 160
K = 125

_pcall = pl.pallas_call


def _sc_segsum(table, sidx, didx, zeros, s_out, s_pad):
    C = table.shape[1]
    dt = table.dtype
    ch = sidx.shape[0] // NW
    stripe = s_pad // NS
    out_tiles = NS if s_out % NS == 0 else 8
    ostripe = s_out // out_tiles
    mesh = plsc.VectorSubcoreMesh(core_axis_name="c", subcore_axis_name="s")

    @functools.partial(
        pl.kernel, mesh=mesh,
        compiler_params=pltpu.CompilerParams(use_tc_tiling_on_sc=False),
        out_type=jax.ShapeDtypeStruct((NC * s_out, CW), dt),
        scratch_types=[
            pltpu.VMEM((ch, K), jnp.int32),
            pltpu.VMEM((ch, K), jnp.int32),
            pltpu.VMEM((K, C), dt),
            pltpu.VMEM((K, C), dt),
            pltpu.VMEM_SHARED((s_pad, C), dt),
            pltpu.SemaphoreType.DMA,
            pltpu.SemaphoreType.DMA,
        ])
    def run(table_h, sidx_h, didx_h, zeros_h, out_h,
            sidx_v, didx_v, rows0, rows1, acc, sem0, sem1):
        c = lax.axis_index("c")
        s = lax.axis_index("s")
        wid = c * NS + s
        pltpu.sync_copy(sidx_h.at[pl.ds(wid * ch, ch)], sidx_v)
        pltpu.sync_copy(didx_h.at[pl.ds(wid * ch, ch)], didx_v)
        pltpu.sync_copy(zeros_h.at[pl.ds(0, stripe)],
                        acc.at[pl.ds(s * stripe, stripe)])
        plsc.subcore_barrier()
        pltpu.async_copy(table_h.at[sidx_v.at[0]], rows0, sem0)

        def step(t, carry):
            j = 2 * t
            pltpu.async_copy(table_h.at[sidx_v.at[j + 1]], rows1, sem1)
            pltpu.make_async_copy(zeros_h.at[pl.ds(0, K)], rows0, sem0).wait()
            pltpu.sync_copy(rows0, acc.at[didx_v.at[j]], add=True)

            @pl.when(t + 1 < ch // 2)
            def _():
                pltpu.async_copy(table_h.at[sidx_v.at[j + 2]], rows0, sem0)

            pltpu.make_async_copy(zeros_h.at[pl.ds(0, K)], rows1, sem1).wait()
            pltpu.sync_copy(rows1, acc.at[didx_v.at[j + 1]], add=True)
            return carry

        lax.fori_loop(0, ch // 2, step, 0)
        plsc.subcore_barrier()

        @pl.when(s < out_tiles)
        def _():
            pltpu.sync_copy(acc.at[pl.ds(s * ostripe, ostripe), pl.ds(0, CW)],
                            out_h.at[pl.ds(c * s_out + s * ostripe, ostripe)])

    return run(table, sidx, didx, zeros)


def _hmask(rows, cols, blk):
    r = lax.broadcasted_iota(jnp.int32, (rows, cols), 0)
    c = lax.broadcasted_iota(jnp.int32, (rows, cols), 1)
    return (r == c // blk).astype(jnp.float32)


def _proj_body(x_ref, w_ref, b_ref, aw_ref, out_ref):
    x = x_ref[...]
    bn = x.shape[0]
    feats, auxs = [], []
    for h in range(4):
        xw = jnp.dot(x, w_ref[h], preferred_element_type=jnp.float32)
        xw = xw + b_ref[h][None, :]
        feats.append(xw)
        auxs.append(jnp.sum(xw * aw_ref[h][None, :], axis=1, keepdims=True))
    ones = jnp.ones((bn, 1), jnp.float32)
    zpad = jnp.zeros((bn, CAUG - 133), jnp.float32)
    out_ref[...] = jnp.concatenate(
        feats + auxs + [ones, zpad], axis=1).astype(jnp.bfloat16)


def _edge_body(p0_ref, p1_ref, out_ref):
    p0 = p0_ref[0].astype(jnp.float32)
    p1 = p1_ref[0].astype(jnp.float32)
    esum = p0[:, :128] + p1[:, :128]
    aux = p0[:, 128:144] + p1[:, 128:144]
    cnt = aux[:, 4:5]
    inv = 1.0 / jnp.maximum(cnt, 1.0)
    alpha128 = jnp.dot(aux, _hmask(16, 128, 32),
                       preferred_element_type=jnp.float32) * inv
    g128 = jnp.exp(jnp.where(alpha128 >= 0.0, alpha128, 0.2 * alpha128))
    yg = g128 * (esum * inv)
    alpha16 = aux * inv
    m16 = (lax.broadcasted_iota(jnp.int32, (1, 16), 1) < 4).astype(jnp.float32)
    g16 = jnp.exp(jnp.where(alpha16 >= 0.0, alpha16, 0.2 * alpha16)) * m16
    out_ref[...] = jnp.concatenate(
        [yg, g16, jnp.zeros((g16.shape[0], CAUG - 144), jnp.float32)],
        axis=1).astype(jnp.bfloat16)


def _final_body(p0_ref, p1_ref, x_ref, convw_ref,
                convb_ref, lng_ref, lnb_ref, gam_ref, out_ref):
    p0 = p0_ref[0].astype(jnp.float32)
    p1 = p1_ref[0].astype(jnp.float32)
    numer = p0[:, :128] + p1[:, :128]
    aux = p0[:, 128:144] + p1[:, 128:144]
    den128 = jnp.dot(aux, _hmask(16, 128, 32),
                     preferred_element_type=jnp.float32) + 1e-12
    o = numer / den128
    o = jnp.where(o > 0.0, o, jnp.exp(jnp.minimum(o, 0.0)) - 1.0)
    mu = jnp.mean(o, axis=1, keepdims=True)
    xc = o - mu
    var = jnp.mean(xc * xc, axis=1, keepdims=True)
    xn = xc * lax.rsqrt(var + 1e-6) * lng_ref[...] + lnb_ref[...]
    xg = 0.5 * xn * (1.0 + lax.erf(xn * 0.7071067811865476))
    xo = jnp.dot(xg, convw_ref[...], preferred_element_type=jnp.float32)
    xo = xo + convb_ref[...]
    out_ref[...] = x_ref[...] + gam_ref[...] * xo


def kernel(X, theta_W, theta_b, atten_e_W, ln_g, ln_b, conv_W, conv_b, gamma,
           v_idx, e_idx):
    N, D = X.shape
    H, _, DH = theta_W.shape
    P = v_idx.shape[0]
    m_pad = ((M + NS * 8 - 1) // (NS * 8)) * (NS * 8)

    f32 = jnp.float32
    bf16 = jnp.bfloat16
    zeros = jnp.zeros((N // NS + 15, CAUG), bf16)
    v2d = v_idx.reshape(P // K, K)
    e2d = e_idx.reshape(P // K, K)

    bn = 1000
    xwaug = _pcall(
        _proj_body,
        grid=(N // bn,),
        in_specs=[
            pl.BlockSpec((bn, D), lambda i: (i, 0)),
            pl.BlockSpec((H, D, DH), lambda i: (0, 0, 0)),
            pl.BlockSpec((H, DH), lambda i: (0, 0)),
            pl.BlockSpec((H, DH), lambda i: (0, 0)),
        ],
        out_specs=pl.BlockSpec((bn, CAUG), lambda i: (i, 0)),
        out_shape=jax.ShapeDtypeStruct((N, CAUG), bf16),
    )(X, theta_W, theta_b, atten_e_W)

    eb = _sc_segsum(xwaug, v2d, e2d, zeros, M, m_pad).reshape(NC, M, CW)

    bm = 1000
    yaug = _pcall(
        _edge_body,
        grid=(M // bm,),
        in_specs=[
            pl.BlockSpec((1, bm, CW), lambda i: (0, i, 0)),
            pl.BlockSpec((1, bm, CW), lambda i: (1, i, 0)),
        ],
        out_specs=pl.BlockSpec((bm, CAUG), lambda i: (i, 0)),
        out_shape=jax.ShapeDtypeStruct((M, CAUG), bf16),
    )(eb, eb)

    vb = _sc_segsum(yaug, e2d, v2d, zeros, N, N).reshape(NC, N, CW)

    out = _pcall(
        _final_body,
        grid=(N // bn,),
        in_specs=[
            pl.BlockSpec((1, bn, CW), lambda i: (0, i, 0)),
            pl.BlockSpec((1, bn, CW), lambda i: (1, i, 0)),
            pl.BlockSpec((bn, D), lambda i: (i, 0)),
            pl.BlockSpec((D, D), lambda i: (0, 0)),
            pl.BlockSpec((1, D), lambda i: (0, 0)),
            pl.BlockSpec((1, D), lambda i: (0, 0)),
            pl.BlockSpec((1, D), lambda i: (0, 0)),
            pl.BlockSpec((1, D), lambda i: (0, 0)),
        ],
        out_specs=pl.BlockSpec((bn, D), lambda i: (i, 0)),
        out_shape=jax.ShapeDtypeStruct((N, D), f32),
    )(vb, vb, X, conv_W, conv_b.reshape(1, D),
      ln_g.reshape(1, D), ln_b.reshape(1, D), gamma.reshape(1, D))
    return out


# speedup vs baseline: 1.0322x; 1.0316x over previous
import functools

import jax
import jax.numpy as jnp
from jax import lax
from jax.experimental import pallas as pl
from jax.experimental.pallas import tpu as pltpu
from jax.experimental.pallas import tpu_sc as plsc

M = 5000
NC, NS = 2, 16
NW = NC * NS
CAUG = 160
K = 125

_pcall = pl.pallas_call


def _sc_segsum(table, sidx, didx, zeros, s_out, s_pad):
    C = table.shape[1]
    dt = table.dtype
    ch = sidx.shape[0] // NW
    stripe = s_pad // NS
    out_tiles = NS if s_out % NS == 0 else 8
    ostripe = s_out // out_tiles
    mesh = plsc.VectorSubcoreMesh(core_axis_name="c", subcore_axis_name="s")

    @functools.partial(
        pl.kernel, mesh=mesh,
        compiler_params=pltpu.CompilerParams(use_tc_tiling_on_sc=False),
        out_type=jax.ShapeDtypeStruct((NC * s_out, C), dt),
        scratch_types=[
            pltpu.VMEM((ch, K), jnp.int32),
            pltpu.VMEM((ch, K), jnp.int32),
            pltpu.VMEM((K, C), dt),
            pltpu.VMEM((K, C), dt),
            pltpu.VMEM_SHARED((s_pad, C), dt),
            pltpu.SemaphoreType.DMA,
            pltpu.SemaphoreType.DMA,
        ])
    def run(table_h, sidx_h, didx_h, zeros_h, out_h,
            sidx_v, didx_v, rows0, rows1, acc, sem0, sem1):
        c = lax.axis_index("c")
        s = lax.axis_index("s")
        wid = c * NS + s
        pltpu.sync_copy(sidx_h.at[pl.ds(wid * ch, ch)], sidx_v)
        pltpu.sync_copy(didx_h.at[pl.ds(wid * ch, ch)], didx_v)
        pltpu.sync_copy(zeros_h.at[pl.ds(0, stripe)],
                        acc.at[pl.ds(s * stripe, stripe)])
        plsc.subcore_barrier()
        pltpu.async_copy(table_h.at[sidx_v.at[0]], rows0, sem0)

        def step(t, carry):
            j = 2 * t
            pltpu.async_copy(table_h.at[sidx_v.at[j + 1]], rows1, sem1)
            pltpu.make_async_copy(zeros_h.at[pl.ds(0, K)], rows0, sem0).wait()
            pltpu.sync_copy(rows0, acc.at[didx_v.at[j]], add=True)

            @pl.when(t + 1 < ch // 2)
            def _():
                pltpu.async_copy(table_h.at[sidx_v.at[j + 2]], rows0, sem0)

            pltpu.make_async_copy(zeros_h.at[pl.ds(0, K)], rows1, sem1).wait()
            pltpu.sync_copy(rows1, acc.at[didx_v.at[j + 1]], add=True)
            return carry

        lax.fori_loop(0, ch // 2, step, 0)
        plsc.subcore_barrier()

        @pl.when(s < out_tiles)
        def _():
            pltpu.sync_copy(acc.at[pl.ds(s * ostripe, ostripe)],
                            out_h.at[pl.ds(c * s_out + s * ostripe, ostripe)])

    return run(table, sidx, didx, zeros)


def _proj_body(x_ref, wcat_ref, bcat_ref, wz_ref, c16_ref, out_ref):
    xw = jnp.dot(x_ref[...], wcat_ref[...], preferred_element_type=jnp.float32)
    xw = xw + bcat_ref[...]
    aux = jnp.dot(xw, wz_ref[...], preferred_element_type=jnp.float32) + c16_ref[...]
    out_ref[...] = jnp.concatenate(
        [xw, aux, jnp.zeros_like(aux)], axis=1).astype(jnp.bfloat16)


def _edge_body(p0_ref, p1_ref, r_ref, m16_ref, out_ref):
    p0 = p0_ref[0].astype(jnp.float32)
    p1 = p1_ref[0].astype(jnp.float32)
    esum = p0[:, :128] + p1[:, :128]
    aux = p0[:, 128:144] + p1[:, 128:144]
    cnt = aux[:, 4:5]
    inv = 1.0 / jnp.maximum(cnt, 1.0)
    alpha128 = jnp.dot(aux, r_ref[...], preferred_element_type=jnp.float32) * inv
    g128 = jnp.exp(jnp.where(alpha128 >= 0.0, alpha128, 0.2 * alpha128))
    yg = g128 * (esum * inv)
    alpha16 = aux * inv
    g16 = jnp.exp(jnp.where(alpha16 >= 0.0, alpha16, 0.2 * alpha16)) * m16_ref[...]
    out_ref[...] = jnp.concatenate(
        [yg, g16, jnp.zeros_like(g16)], axis=1).astype(jnp.bfloat16)


def _final_body(p0_ref, p1_ref, x_ref, r_ref, convw_ref,
                convb_ref, lng_ref, lnb_ref, gam_ref, out_ref):
    p0 = p0_ref[0].astype(jnp.float32)
    p1 = p1_ref[0].astype(jnp.float32)
    numer = p0[:, :128] + p1[:, :128]
    aux = p0[:, 128:144] + p1[:, 128:144]
    den128 = jnp.dot(aux, r_ref[...], preferred_element_type=jnp.float32) + 1e-12
    o = numer / den128
    o = jnp.where(o > 0.0, o, jnp.exp(jnp.minimum(o, 0.0)) - 1.0)
    mu = jnp.mean(o, axis=1, keepdims=True)
    xc = o - mu
    var = jnp.mean(xc * xc, axis=1, keepdims=True)
    xn = xc * lax.rsqrt(var + 1e-6) * lng_ref[...] + lnb_ref[...]
    xg = 0.5 * xn * (1.0 + lax.erf(xn * 0.7071067811865476))
    xo = jnp.dot(xg, convw_ref[...], preferred_element_type=jnp.float32)
    xo = xo + convb_ref[...]
    out_ref[...] = x_ref[...] + gam_ref[...] * xo


def kernel(X, theta_W, theta_b, atten_e_W, ln_g, ln_b, conv_W, conv_b, gamma,
           v_idx, e_idx):
    N, D = X.shape
    H, _, DH = theta_W.shape
    P = v_idx.shape[0]
    m_pad = ((M + NS * 8 - 1) // (NS * 8)) * (NS * 8)

    f32 = jnp.float32
    bf16 = jnp.bfloat16
    Wcat = jnp.transpose(theta_W, (1, 0, 2)).reshape(D, H * DH)
    bcat = theta_b.reshape(1, H * DH)
    Wz = (jnp.eye(H, dtype=f32)[:, None, :] * atten_e_W[:, :, None]).reshape(H * DH, H)
    Wz16 = jnp.pad(Wz, ((0, 0), (0, 16 - H)))
    c16 = jnp.zeros((1, 16), f32).at[0, H].set(1.0)
    R = jnp.pad(jnp.repeat(jnp.eye(H, dtype=f32), DH, axis=1), ((0, 16 - H), (0, 0)))
    m16 = jnp.zeros((1, 16), f32).at[0, :H].set(1.0)
    zeros = jnp.zeros((N // NS + 15, CAUG), bf16)
    v2d = v_idx.reshape(P // K, K)
    e2d = e_idx.reshape(P // K, K)

    bn = 1000
    xwaug = _pcall(
        _proj_body,
        grid=(N // bn,),
        in_specs=[
            pl.BlockSpec((bn, D), lambda i: (i, 0)),
            pl.BlockSpec((D, D), lambda i: (0, 0)),
            pl.BlockSpec((1, D), lambda i: (0, 0)),
            pl.BlockSpec((D, 16), lambda i: (0, 0)),
            pl.BlockSpec((1, 16), lambda i: (0, 0)),
        ],
        out_specs=pl.BlockSpec((bn, CAUG), lambda i: (i, 0)),
        out_shape=jax.ShapeDtypeStruct((N, CAUG), bf16),
    )(X, Wcat, bcat, Wz16, c16)

    eb = _sc_segsum(xwaug, v2d, e2d, zeros, M, m_pad).reshape(NC, M, CAUG)

    bm = 1000
    yaug = _pcall(
        _edge_body,
        grid=(M // bm,),
        in_specs=[
            pl.BlockSpec((1, bm, CAUG), lambda i: (0, i, 0)),
            pl.BlockSpec((1, bm, CAUG), lambda i: (1, i, 0)),
            pl.BlockSpec((16, D), lambda i: (0, 0)),
            pl.BlockSpec((1, 16), lambda i: (0, 0)),
        ],
        out_specs=pl.BlockSpec((bm, CAUG), lambda i: (i, 0)),
        out_shape=jax.ShapeDtypeStruct((M, CAUG), bf16),
    )(eb, eb, R, m16)

    vb = _sc_segsum(yaug, e2d, v2d, zeros, N, N).reshape(NC, N, CAUG)

    out = _pcall(
        _final_body,
        grid=(N // bn,),
        in_specs=[
            pl.BlockSpec((1, bn, CAUG), lambda i: (0, i, 0)),
            pl.BlockSpec((1, bn, CAUG), lambda i: (1, i, 0)),
            pl.BlockSpec((bn, D), lambda i: (i, 0)),
            pl.BlockSpec((16, D), lambda i: (0, 0)),
            pl.BlockSpec((D, D), lambda i: (0, 0)),
            pl.BlockSpec((1, D), lambda i: (0, 0)),
            pl.BlockSpec((1, D), lambda i: (0, 0)),
            pl.BlockSpec((1, D), lambda i: (0, 0)),
            pl.BlockSpec((1, D), lambda i: (0, 0)),
        ],
        out_specs=pl.BlockSpec((bn, D), lambda i: (i, 0)),
        out_shape=jax.ShapeDtypeStruct((N, D), f32),
    )(vb, vb, X, R, conv_W, conv_b.reshape(1, D),
      ln_g.reshape(1, D), ln_b.reshape(1, D), gamma.reshape(1, D))
    return out
